# baseline (speedup 1.0000x reference)
import functools

import jax
import jax.numpy as jnp
from jax import lax
from jax.experimental import pallas as pl
from jax.experimental.pallas import tpu as pltpu
from jax.experimental.pallas import tpu_sc as plsc

_B = 16384
_D = 32
_L = 16
_NC = 2
_NS = 16
_NW = _NC * _NS
_BPW = _B // _NW
_HC = 16
_RC = _L * _HC
_V = 1000000

_mesh = plsc.VectorSubcoreMesh(core_axis_name="c", subcore_axis_name="s")


def _body(ub_hbm, ib_hbm, ut_hbm, it_hbm, out_hbm,
          uidx_v, iidx_v, us_v, is_v, ubuf, ibuf, out_v, sem_u, sem_i):
    wid = lax.axis_index("s") * _NC + lax.axis_index("c")
    base = wid * _BPW

    pltpu.sync_copy(ub_hbm.at[pl.ds(base, _BPW)], uidx_v)
    pltpu.sync_copy(ib_hbm.at[pl.ds(base, _BPW)], iidx_v)

    lanes = lax.iota(jnp.int32, _L)

    def chunk_body(ch, carry):
        ru = uidx_v[pl.ds(ch * _L, _L)]
        ri = iidx_v[pl.ds(ch * _L, _L)]
        acc = jnp.zeros((_L,), jnp.float32)
        for half in range(_D // _HC):
            c0 = half * _HC
            for cl in range(_HC):
                c = c0 + cl
                us_v[pl.ds(cl * _L, _L)] = (ru + c * _V) >> 7
                is_v[pl.ds(cl * _L, _L)] = (ri + c * _V) >> 7
            cu = pltpu.async_copy(ut_hbm.at[us_v], ubuf, sem_u)
            ci = pltpu.async_copy(it_hbm.at[is_v], ibuf, sem_i)
            cu.wait()
            ci.wait()
            for cl in range(_HC):
                c = c0 + cl
                slots = cl * _L + lanes
                offu = (ru + c * 64) & 127
                offi = (ri + c * 64) & 127
                uu = plsc.load_gather(ubuf, [slots, offu])
                vv = plsc.load_gather(ibuf, [slots, offi])
                acc = acc + uu * vv
        out_v[pl.ds(ch * _L, _L)] = acc
        return carry

    lax.fori_loop(0, _BPW // _L, chunk_body, 0)

    pltpu.sync_copy(out_v, out_hbm.at[pl.ds(base, _BPW)])


@jax.jit
def _run(user_batch, item_batch, ut128, it128):
    k = functools.partial(
        pl.kernel,
        out_type=jax.ShapeDtypeStruct((_B,), jnp.float32),
        mesh=_mesh,
        scratch_types=[
            pltpu.VMEM((_BPW,), jnp.int32),
            pltpu.VMEM((_BPW,), jnp.int32),
            pltpu.VMEM((_RC,), jnp.int32),
            pltpu.VMEM((_RC,), jnp.int32),
            pltpu.VMEM((_RC, 128), jnp.float32),
            pltpu.VMEM((_RC, 128), jnp.float32),
            pltpu.VMEM((_BPW,), jnp.float32),
            pltpu.SemaphoreType.DMA,
            pltpu.SemaphoreType.DMA,
        ],
        compiler_params=pltpu.CompilerParams(needs_layout_passes=False),
    )(_body)
    return k(user_batch, item_batch, ut128, it128)


def kernel(user_batch, item_batch, user_table, item_table):
    return _run(user_batch.astype(jnp.int32), item_batch.astype(jnp.int32),
                user_table.T.reshape(250000, 128),
                item_table.T.reshape(250000, 128))


# speedup vs baseline: 8.1267x; 8.1267x over previous
import functools

import jax
import jax.numpy as jnp
from jax import lax
from jax.experimental import pallas as pl
from jax.experimental.pallas import tpu as pltpu
from jax.experimental.pallas import tpu_sc as plsc

_B = 16384
_D = 32
_L = 16
_NC = 2
_NS = 16
_NW = _NC * _NS
_BPW = _B // _NW
_CH = 256
_NCH = _BPW // _CH

_mesh = plsc.VectorSubcoreMesh(core_axis_name="c", subcore_axis_name="s")


def _body(ub_hbm, ib_hbm, ut_hbm, it_hbm, out_hbm,
          uidx_v, iidx_v, us_v, is_v, ubuf, ibuf, out_v, sem_u, sem_i):
    wid = lax.axis_index("s") * _NC + lax.axis_index("c")
    base = wid * _BPW

    pltpu.sync_copy(ub_hbm.at[pl.ds(base, _BPW)], uidx_v)
    pltpu.sync_copy(ib_hbm.at[pl.ds(base, _BPW)], iidx_v)

    lanes = lax.iota(jnp.int32, _L)
    cols = [(j + lanes) & (_D - 1) for j in range(_D)]

    def chunk_body(c, carry):
        cb = c * _CH

        def mk_super(i, carry2):
            us_v[pl.ds(i * _L, _L)] = uidx_v[pl.ds(cb + i * _L, _L)] >> 2
            is_v[pl.ds(i * _L, _L)] = iidx_v[pl.ds(cb + i * _L, _L)] >> 2
            return carry2

        lax.fori_loop(0, _CH // _L, mk_super, 0)

        cu = pltpu.async_copy(ut_hbm.at[us_v], ubuf, sem_u)
        ci = pltpu.async_copy(it_hbm.at[is_v], ibuf, sem_i)
        cu.wait()
        ci.wait()

        def group(g, carry2):
            b = cb + g * _L
            slots = g * _L + lanes
            ucb = (uidx_v[pl.ds(b, _L)] & 3) * _D
            icb = (iidx_v[pl.ds(b, _L)] & 3) * _D
            acc = jnp.zeros((_L,), jnp.float32)
            for j in range(_D):
                uu = plsc.load_gather(ubuf, [slots, ucb + cols[j]])
                vv = plsc.load_gather(ibuf, [slots, icb + cols[j]])
                acc = acc + uu * vv
            out_v[pl.ds(b, _L)] = acc
            return carry2

        lax.fori_loop(0, _CH // _L, group, 0)
        return carry

    lax.fori_loop(0, _NCH, chunk_body, 0)

    pltpu.sync_copy(out_v, out_hbm.at[pl.ds(base, _BPW)])


@jax.jit
def _run(user_batch, item_batch, user_table, item_table):
    k = functools.partial(
        pl.kernel,
        out_type=jax.ShapeDtypeStruct((_B,), jnp.float32),
        mesh=_mesh,
        scratch_types=[
            pltpu.VMEM((_BPW,), jnp.int32),
            pltpu.VMEM((_BPW,), jnp.int32),
            pltpu.VMEM((_CH,), jnp.int32),
            pltpu.VMEM((_CH,), jnp.int32),
            pltpu.VMEM((_CH, 4 * _D), jnp.float32),
            pltpu.VMEM((_CH, 4 * _D), jnp.float32),
            pltpu.VMEM((_BPW,), jnp.float32),
            pltpu.SemaphoreType.DMA,
            pltpu.SemaphoreType.DMA,
        ],
        compiler_params=pltpu.CompilerParams(needs_layout_passes=False),
    )(_body)
    ut128 = user_table.reshape(-1, 4 * _D)
    it128 = item_table.reshape(-1, 4 * _D)
    return k(user_batch, item_batch, ut128, it128)


def kernel(user_batch, item_batch, user_table, item_table):
    return _run(user_batch.astype(jnp.int32), item_batch.astype(jnp.int32),
                user_table, item_table)
